# Initial kernel scaffold; baseline (speedup 1.0000x reference)
import functools

import jax
import jax.numpy as jnp
from jax import lax
from jax.experimental import pallas as pl
from jax.experimental.pallas import tpu as pltpu
from jax.experimental.pallas import tpu_sc as plsc

N_NODES = 10000
N_EDGES = 320000
FEAT = 128
N_GRAPHS = 64

BLK = 128
R_PAD = 10112
NBLK = R_PAD // BLK

NC = 2
NSUB = 16
NW = NC * NSUB
CHUNK = 80
EPW = N_EDGES // NW
NCHUNK = EPW // CHUNK
RPS = R_PAD // NSUB

_mesh = plsc.VectorSubcoreMesh(core_axis_name="c", subcore_axis_name="s")


@functools.partial(
    pl.kernel,
    out_type=jax.ShapeDtypeStruct((NC, R_PAD), jnp.float32),
    mesh=_mesh,
    scratch_types=[
        pltpu.VMEM((1, CHUNK), jnp.int32),
        pltpu.VMEM((CHUNK,), jnp.float32),
        pltpu.VMEM_SHARED((R_PAD,), jnp.float32),
    ],
)
def _degree_sc(dst_hbm, ones_hbm, z1_hbm, out_hbm, dbuf, ones_v, dacc):
    c = lax.axis_index("c")
    s = lax.axis_index("s")
    wid = c * NSUB + s
    pltpu.sync_copy(z1_hbm, dacc.at[pl.ds(s * RPS, RPS)])
    pltpu.sync_copy(ones_hbm, ones_v)
    plsc.subcore_barrier()
    base = wid * EPW

    @pl.loop(0, NCHUNK)
    def _(j):
        pltpu.sync_copy(dst_hbm.at[pl.ds(base + j * CHUNK, CHUNK)], dbuf.at[0])
        pltpu.sync_copy(ones_v, dacc.at[dbuf.at[0]], add=True)

    plsc.subcore_barrier()
    pltpu.sync_copy(dacc.at[pl.ds(s * RPS, RPS)],
                    out_hbm.at[c, pl.ds(s * RPS, RPS)])


@functools.partial(
    pl.kernel,
    out_type=jax.ShapeDtypeStruct((NC, R_PAD, FEAT), jnp.float32),
    mesh=_mesh,
    scratch_types=[
        pltpu.VMEM((1, CHUNK), jnp.int32),
        pltpu.VMEM((1, CHUNK), jnp.int32),
        pltpu.VMEM((1, CHUNK, FEAT), jnp.float32),
        pltpu.VMEM_SHARED((R_PAD, FEAT), jnp.float32),
    ],
)
def _propagate_sc(g_hbm, src_hbm, dst_hbm, z_hbm, out_hbm, sbuf, dbuf, rows, acc):
    c = lax.axis_index("c")
    s = lax.axis_index("s")
    wid = c * NSUB + s
    pltpu.sync_copy(z_hbm, acc.at[pl.ds(s * RPS, RPS)])
    plsc.subcore_barrier()
    base = wid * EPW

    @pl.loop(0, NCHUNK)
    def _(j):
        off = base + j * CHUNK
        pltpu.sync_copy(src_hbm.at[pl.ds(off, CHUNK)], sbuf.at[0])
        pltpu.sync_copy(dst_hbm.at[pl.ds(off, CHUNK)], dbuf.at[0])
        pltpu.sync_copy(g_hbm.at[sbuf.at[0]], rows.at[0])
        pltpu.sync_copy(rows.at[0], acc.at[dbuf.at[0]], add=True)

    plsc.subcore_barrier()
    pltpu.sync_copy(acc.at[pl.ds(s * RPS, RPS)],
                    out_hbm.at[c, pl.ds(s * RPS, RPS)])


def _c1_body(x_ref, d0_ref, d1_ref, w_ref, g_ref):
    dv = lax.rsqrt(d0_ref[...] + d1_ref[...] + 1.0)
    g_ref[...] = jnp.dot(x_ref[...], w_ref[...],
                         preferred_element_type=jnp.float32) * dv


def _cmid_body(a0_ref, a1_ref, gp_ref, d0_ref, d1_ref, b_ref, w_ref, oh_ref,
               g_ref, pool_ref):
    dv = lax.rsqrt(d0_ref[...] + d1_ref[...] + 1.0)
    act = jnp.maximum((a0_ref[...] + a1_ref[...] + gp_ref[...]) * dv
                      + b_ref[...], 0.0)

    @pl.when(pl.program_id(0) == 0)
    def _():
        pool_ref[...] = jnp.zeros_like(pool_ref)

    pool_ref[...] += lax.dot_general(
        oh_ref[...], act, (((0,), (0,)), ((), ())),
        preferred_element_type=jnp.float32)
    g_ref[...] = jnp.dot(act, w_ref[...],
                         preferred_element_type=jnp.float32) * dv


def _cfin_body(a0_ref, a1_ref, gp_ref, d0_ref, d1_ref, b_ref, oh_ref,
               pool_ref):
    dv = lax.rsqrt(d0_ref[...] + d1_ref[...] + 1.0)
    act = jnp.maximum((a0_ref[...] + a1_ref[...] + gp_ref[...]) * dv
                      + b_ref[...], 0.0)

    @pl.when(pl.program_id(0) == 0)
    def _():
        pool_ref[...] = jnp.zeros_like(pool_ref)

    pool_ref[...] += lax.dot_general(
        oh_ref[...], act, (((0,), (0,)), ((), ())),
        preferred_element_type=jnp.float32)


_row_spec = pl.BlockSpec((BLK, FEAT), lambda m: (m, 0))
_col_spec = pl.BlockSpec((BLK, 1), lambda m: (m, 0))
_w_spec = pl.BlockSpec((FEAT, FEAT), lambda m: (0, 0))
_b_spec = pl.BlockSpec((1, FEAT), lambda m: (0, 0))
_oh_spec = pl.BlockSpec((BLK, N_GRAPHS), lambda m: (m, 0))
_pool_spec = pl.BlockSpec((N_GRAPHS, FEAT), lambda m: (0, 0))


def _c1_tc(x_pad, d0, d1, W):
    return pl.pallas_call(
        _c1_body,
        grid=(NBLK,),
        in_specs=[_row_spec, _col_spec, _col_spec, _w_spec],
        out_specs=_row_spec,
        out_shape=jax.ShapeDtypeStruct((R_PAD, FEAT), jnp.float32),
    )(x_pad, d0, d1, W)


def _cmid_tc(a0, a1, gp, d0, d1, b, W, oh):
    return pl.pallas_call(
        _cmid_body,
        grid=(NBLK,),
        in_specs=[_row_spec, _row_spec, _row_spec, _col_spec, _col_spec,
                  _b_spec, _w_spec, _oh_spec],
        out_specs=[_row_spec, _pool_spec],
        out_shape=[jax.ShapeDtypeStruct((R_PAD, FEAT), jnp.float32),
                   jax.ShapeDtypeStruct((N_GRAPHS, FEAT), jnp.float32)],
    )(a0, a1, gp, d0, d1, b, W, oh)


def _cfin_tc(a0, a1, gp, d0, d1, b, oh):
    return pl.pallas_call(
        _cfin_body,
        grid=(NBLK,),
        in_specs=[_row_spec, _row_spec, _row_spec, _col_spec, _col_spec,
                  _b_spec, _oh_spec],
        out_specs=_pool_spec,
        out_shape=jax.ShapeDtypeStruct((N_GRAPHS, FEAT), jnp.float32),
    )(a0, a1, gp, d0, d1, b, oh)


def kernel(x, edge_index, batch, W1, b1, W2, b2, W3, b3):
    src = edge_index[0].astype(jnp.int32)
    dst = edge_index[1].astype(jnp.int32)

    x_pad = jnp.pad(x, ((0, R_PAD - N_NODES), (0, 0)))
    oh = (batch[:, None] == jnp.arange(N_GRAPHS, dtype=batch.dtype)[None, :])
    oh = jnp.pad(oh.astype(jnp.float32), ((0, R_PAD - N_NODES), (0, 0)))
    ones_c = jnp.ones((CHUNK,), jnp.float32)
    zeros_1d = jnp.zeros((RPS,), jnp.float32)
    zeros_blk = jnp.zeros((RPS, FEAT), jnp.float32)
    b1r = b1.reshape(1, FEAT)
    b2r = b2.reshape(1, FEAT)
    b3r = b3.reshape(1, FEAT)

    deg = _degree_sc(dst, ones_c, zeros_1d)
    d0 = deg[0].reshape(R_PAD, 1)
    d1 = deg[1].reshape(R_PAD, 1)

    g1 = _c1_tc(x_pad, d0, d1, W1)
    a1 = _propagate_sc(g1, src, dst, zeros_blk)
    g2, pool1 = _cmid_tc(a1[0], a1[1], g1, d0, d1, b1r, W2, oh)
    a2 = _propagate_sc(g2, src, dst, zeros_blk)
    g3, pool2 = _cmid_tc(a2[0], a2[1], g2, d0, d1, b2r, W3, oh)
    a3 = _propagate_sc(g3, src, dst, zeros_blk)
    pool3 = _cfin_tc(a3[0], a3[1], g3, d0, d1, b3r, oh)

    return jnp.concatenate([pool1, pool2, pool3], axis=1)


# speedup vs baseline: 9.7664x; 9.7664x over previous
import functools

import jax
import jax.numpy as jnp
from jax import lax
from jax.experimental import pallas as pl
from jax.experimental.pallas import tpu as pltpu
from jax.experimental.pallas import tpu_sc as plsc

N_NODES = 10000
N_EDGES = 320000
FEAT = 128
N_GRAPHS = 64

BLK = 128
R_PAD = 10240
NBLK = R_PAD // BLK

NC = 2
NSUB = 16
NW = NC * NSUB
CHUNK = 80
EPW = N_EDGES // NW
NCHUNK = EPW // CHUNK
RPS = R_PAD // NSUB

@functools.cache
def _sc_kernels():
    mesh = plsc.VectorSubcoreMesh(core_axis_name="c", subcore_axis_name="s")

    @functools.partial(
        pl.kernel,
        out_type=jax.ShapeDtypeStruct((NC * R_PAD,), jnp.float32),
        mesh=mesh,
        scratch_types=[
            pltpu.VMEM((1, CHUNK), jnp.int32),
            pltpu.VMEM((CHUNK,), jnp.float32),
            pltpu.VMEM((RPS,), jnp.float32),
            pltpu.VMEM_SHARED((R_PAD,), jnp.float32),
        ],
    )
    def degree_sc(dst_hbm, out_hbm, dbuf, ones_v, zbuf, dacc):
        c = lax.axis_index("c")
        s = lax.axis_index("s")
        wid = c * NSUB + s

        @pl.loop(0, CHUNK // 16)
        def _(i):
            ones_v[pl.ds(i * 16, 16)] = jnp.ones((16,), jnp.float32)

        @pl.loop(0, RPS // 16)
        def _(i):
            zbuf[pl.ds(i * 16, 16)] = jnp.zeros((16,), jnp.float32)

        pltpu.sync_copy(zbuf, dacc.at[pl.ds(s * RPS, RPS)])
        plsc.subcore_barrier()
        base = wid * EPW

        @pl.loop(0, NCHUNK)
        def _(j):
            pltpu.sync_copy(dst_hbm.at[pl.ds(base + j * CHUNK, CHUNK)],
                            dbuf.at[0])
            pltpu.sync_copy(ones_v, dacc.at[dbuf.at[0]], add=True)

        plsc.subcore_barrier()
        pltpu.sync_copy(dacc.at[pl.ds(s * RPS, RPS)],
                        out_hbm.at[pl.ds(c * R_PAD + s * RPS, RPS)])

    @functools.partial(
        pl.kernel,
        out_type=jax.ShapeDtypeStruct((NC, R_PAD, FEAT), jnp.float32),
        mesh=mesh,
        scratch_types=[
            pltpu.VMEM((1, CHUNK), jnp.int32),
            pltpu.VMEM((1, CHUNK), jnp.int32),
            pltpu.VMEM((1, CHUNK, FEAT), jnp.float32),
            pltpu.VMEM((BLK, FEAT), jnp.float32),
            pltpu.VMEM_SHARED((R_PAD, FEAT), jnp.float32),
        ],
    )
    def propagate_sc(g_hbm, src_hbm, dst_hbm, out_hbm,
                     sbuf, dbuf, rows, zrow, acc):
        c = lax.axis_index("c")
        s = lax.axis_index("s")
        wid = c * NSUB + s

        @pl.loop(0, BLK)
        def _(i):
            @pl.loop(0, FEAT // 16)
            def _(j):
                zrow[i, pl.ds(j * 16, 16)] = jnp.zeros((16,), jnp.float32)

        @pl.loop(0, RPS // BLK)
        def _(k):
            pltpu.sync_copy(zrow, acc.at[pl.ds(s * RPS + k * BLK, BLK)])

        plsc.subcore_barrier()
        base = wid * EPW

        @pl.loop(0, NCHUNK)
        def _(j):
            off = base + j * CHUNK
            pltpu.sync_copy(src_hbm.at[pl.ds(off, CHUNK)], sbuf.at[0])
            pltpu.sync_copy(dst_hbm.at[pl.ds(off, CHUNK)], dbuf.at[0])
            pltpu.sync_copy(g_hbm.at[sbuf.at[0]], rows.at[0])
            pltpu.sync_copy(rows.at[0], acc.at[dbuf.at[0]], add=True)

        plsc.subcore_barrier()
        pltpu.sync_copy(acc.at[pl.ds(s * RPS, RPS)],
                        out_hbm.at[c, pl.ds(s * RPS, RPS)])

    return degree_sc, propagate_sc


def _c1_body(x_ref, d0_ref, d1_ref, w_ref, g_ref):
    dv = lax.rsqrt(d0_ref[...] + d1_ref[...] + 1.0)
    g_ref[...] = jnp.dot(x_ref[...], w_ref[...],
                         preferred_element_type=jnp.float32) * dv


def _cmid_body(a0_ref, a1_ref, gp_ref, d0_ref, d1_ref, b_ref, w_ref, oh_ref,
               g_ref, pool_ref):
    dv = lax.rsqrt(d0_ref[...] + d1_ref[...] + 1.0)
    act = jnp.maximum((a0_ref[...] + a1_ref[...] + gp_ref[...]) * dv
                      + b_ref[...], 0.0)

    @pl.when(pl.program_id(0) == 0)
    def _():
        pool_ref[...] = jnp.zeros_like(pool_ref)

    pool_ref[...] += lax.dot_general(
        oh_ref[...], act, (((0,), (0,)), ((), ())),
        preferred_element_type=jnp.float32)
    g_ref[...] = jnp.dot(act, w_ref[...],
                         preferred_element_type=jnp.float32) * dv


def _cfin_body(a0_ref, a1_ref, gp_ref, d0_ref, d1_ref, b_ref, oh_ref,
               pool_ref):
    dv = lax.rsqrt(d0_ref[...] + d1_ref[...] + 1.0)
    act = jnp.maximum((a0_ref[...] + a1_ref[...] + gp_ref[...]) * dv
                      + b_ref[...], 0.0)

    @pl.when(pl.program_id(0) == 0)
    def _():
        pool_ref[...] = jnp.zeros_like(pool_ref)

    pool_ref[...] += lax.dot_general(
        oh_ref[...], act, (((0,), (0,)), ((), ())),
        preferred_element_type=jnp.float32)


_row_spec = pl.BlockSpec((BLK, FEAT), lambda m: (m, 0))
_col_spec = pl.BlockSpec((BLK, 1), lambda m: (m, 0))
_w_spec = pl.BlockSpec((FEAT, FEAT), lambda m: (0, 0))
_b_spec = pl.BlockSpec((1, FEAT), lambda m: (0, 0))
_oh_spec = pl.BlockSpec((BLK, N_GRAPHS), lambda m: (m, 0))
_pool_spec = pl.BlockSpec((N_GRAPHS, FEAT), lambda m: (0, 0))


def _c1_tc(x_pad, d0, d1, W):
    return pl.pallas_call(
        _c1_body,
        grid=(NBLK,),
        in_specs=[_row_spec, _col_spec, _col_spec, _w_spec],
        out_specs=_row_spec,
        out_shape=jax.ShapeDtypeStruct((R_PAD, FEAT), jnp.float32),
    )(x_pad, d0, d1, W)


def _cmid_tc(a0, a1, gp, d0, d1, b, W, oh):
    return pl.pallas_call(
        _cmid_body,
        grid=(NBLK,),
        in_specs=[_row_spec, _row_spec, _row_spec, _col_spec, _col_spec,
                  _b_spec, _w_spec, _oh_spec],
        out_specs=[_row_spec, _pool_spec],
        out_shape=[jax.ShapeDtypeStruct((R_PAD, FEAT), jnp.float32),
                   jax.ShapeDtypeStruct((N_GRAPHS, FEAT), jnp.float32)],
    )(a0, a1, gp, d0, d1, b, W, oh)


def _cfin_tc(a0, a1, gp, d0, d1, b, oh):
    return pl.pallas_call(
        _cfin_body,
        grid=(NBLK,),
        in_specs=[_row_spec, _row_spec, _row_spec, _col_spec, _col_spec,
                  _b_spec, _oh_spec],
        out_specs=_pool_spec,
        out_shape=jax.ShapeDtypeStruct((N_GRAPHS, FEAT), jnp.float32),
    )(a0, a1, gp, d0, d1, b, oh)


def kernel(x, edge_index, batch, W1, b1, W2, b2, W3, b3):
    src = edge_index[0].astype(jnp.int32)
    dst = edge_index[1].astype(jnp.int32)

    x_pad = jnp.pad(x, ((0, R_PAD - N_NODES), (0, 0)))
    oh = (batch[:, None] == jnp.arange(N_GRAPHS, dtype=batch.dtype)[None, :])
    oh = jnp.pad(oh.astype(jnp.float32), ((0, R_PAD - N_NODES), (0, 0)))
    b1r = b1.reshape(1, FEAT)
    b2r = b2.reshape(1, FEAT)
    b3r = b3.reshape(1, FEAT)

    degree_sc, propagate_sc = _sc_kernels()
    deg = degree_sc(dst).reshape(NC, R_PAD)
    d0 = deg[0].reshape(R_PAD, 1)
    d1 = deg[1].reshape(R_PAD, 1)

    g1 = _c1_tc(x_pad, d0, d1, W1)
    a1 = propagate_sc(g1, src, dst)
    g2, pool1 = _cmid_tc(a1[0], a1[1], g1, d0, d1, b1r, W2, oh)
    a2 = propagate_sc(g2, src, dst)
    g3, pool2 = _cmid_tc(a2[0], a2[1], g2, d0, d1, b2r, W3, oh)
    a3 = propagate_sc(g3, src, dst)
    pool3 = _cfin_tc(a3[0], a3[1], g3, d0, d1, b3r, oh)

    return jnp.concatenate([pool1, pool2, pool3], axis=1)
